# baseline (speedup 1.0000x reference)
import dataclasses
import functools

import jax
import jax.numpy as jnp
from jax import lax
from jax.experimental import pallas as pl
from jax.experimental.pallas import tpu as pltpu
from jax.experimental.pallas import tpu_sc as plsc

NN = 10000
EE = 320000
DD = 128
NC = 2
NS = 16
NW = NC * NS
EPW = EE // NW
CH = 80
NCHUNK = EPW // CH
NPAD = 10240
RPT = NPAD // NS
NPADM = 10112
RPTM = NPADM // NS

_vmesh = plsc.VectorSubcoreMesh(core_axis_name="c", subcore_axis_name="s")

_sc_params = pltpu.CompilerParams()
if "needs_layout_passes" in pltpu.CompilerParams.__dataclass_fields__:
    _sc_params = dataclasses.replace(_sc_params, needs_layout_passes=False)


def _splat16(v):
    return jnp.zeros((16,), jnp.int32) + v


def _vcopy_idx(src1d, base, dst1d, n):
    q = 0
    while q + 16 <= n:
        dst1d[pl.ds(q, 16)] = src1d[pl.ds(base + q, 16)]
        q += 16
    if q < n:
        dst1d[pl.ds(n - 16, 16)] = src1d[pl.ds(base + n - 16, 16)]


def _sc_deg_body(col_hbm, ew_hbm, out_hbm, colall, ewall, zv, degsh, sem):
    cid = lax.axis_index("c")
    sid = lax.axis_index("s")
    wid = sid * NC + cid

    @pl.loop(0, RPT, step=16)
    def _zero(i):
        zv[pl.ds(i, 16)] = jnp.zeros((16,), jnp.float32)

    pltpu.sync_copy(zv, degsh.at[pl.ds(sid * RPT, RPT)])
    pltpu.sync_copy(col_hbm.at[wid], colall)
    pltpu.sync_copy(ew_hbm.at[wid], ewall)
    plsc.subcore_barrier()

    @pl.loop(0, NCHUNK // 5)
    def _chunk(g):
        descs = []
        for u in range(5):
            c = g * 5 + u
            descs.append(pltpu.async_copy(
                ewall.at[c], degsh.at[colall.at[c]], sem, add=True))
        for d in descs:
            d.wait()

    plsc.subcore_barrier()
    pltpu.sync_copy(degsh.at[pl.ds(sid * RPT, RPT)],
                    out_hbm.at[cid, pl.ds(sid * RPT, RPT)])


def _sc_deg(col3, ew3):
    k = pl.kernel(
        _sc_deg_body,
        out_type=jax.ShapeDtypeStruct((NC, NPAD), jnp.float32),
        mesh=_vmesh,
        scratch_types=[
            pltpu.VMEM((NCHUNK, CH), jnp.int32),
            pltpu.VMEM((NCHUNK, CH), jnp.float32),
            pltpu.VMEM((RPT,), jnp.float32),
            pltpu.VMEM_SHARED((NPAD,), jnp.float32),
            pltpu.SemaphoreType.DMA,
        ],
        compiler_params=_sc_params,
    )
    return k(col3, ew3)


CHM = 80
NCHM = EPW // CHM


def _sc_msg_body(hp_hbm, eib_hbm, out_hbm,
                 ibufs, scolvs, bufs, ssh, gsems, ssems, isems):
    cid = lax.axis_index("c")
    sid = lax.axis_index("s")
    wid = sid * NC + cid

    for b in range(4):
        @pl.loop(0, CHM)
        def _zero(r, _b=b):
            for p in range(DD // 16):
                bufs[_b][r, pl.ds(p * 16, 16)] = jnp.zeros((16,), jnp.float32)

        for q in range(CHM // 16):
            scolvs[b][pl.ds(q * 16, 16)] = jnp.zeros((16,), jnp.int32)

    nz = RPTM // CHM
    rem = RPTM % CHM
    for z in range(nz):
        pltpu.async_copy(bufs[0], ssh.at[pl.ds(sid * RPTM + z * CHM, CHM)],
                         isems[0])
    if rem:
        pltpu.async_copy(
            bufs[0].at[pl.ds(0, rem)],
            ssh.at[pl.ds(sid * RPTM + nz * CHM, rem)], isems[0])
    for z in range(nz):
        pltpu.make_async_copy(
            bufs[0], ssh.at[pl.ds(sid * RPTM + z * CHM, CHM)],
            isems[0]).wait()
    if rem:
        pltpu.make_async_copy(
            bufs[0].at[pl.ds(0, rem)],
            ssh.at[pl.ds(sid * RPTM + nz * CHM, rem)], isems[0]).wait()
    plsc.subcore_barrier()

    ibase = wid * NCHM * 3 * CHM

    def iload(c, k):
        pltpu.async_copy(eib_hbm.at[pl.ds(ibase + c * 3 * CHM, 3 * CHM)],
                         ibufs[k], isems[k])

    def iwait(c, k):
        pltpu.make_async_copy(
            eib_hbm.at[pl.ds(ibase + c * 3 * CHM, 3 * CHM)],
            ibufs[k], isems[k]).wait()

    def gather(k):
        pltpu.async_copy(hp_hbm.at[ibufs[k].at[pl.ds(0, CHM)]],
                         bufs[k], gsems[k])

    def gwait(k):
        pltpu.make_async_copy(hp_hbm.at[ibufs[k].at[pl.ds(0, CHM)]],
                              bufs[k], gsems[k]).wait()

    def swait(k):
        pltpu.make_async_copy(bufs[k], ssh.at[scolvs[k]], ssems[k]).wait()

    def scale(buf, ibuf):
        @pl.loop(0, CHM, step=16)
        def _s(j):
            wv = plsc.bitcast(ibuf[pl.ds(2 * CHM + j, 16)], jnp.float32)
            for u in range(16):
                w = jnp.take(wv, jnp.full((16,), u, jnp.int32))
                for p in range(DD // 16):
                    sl = pl.ds(p * 16, 16)
                    buf[j + u, sl] = buf[j + u, sl] * w

    def quarter(c, k, do_gather, do_iload):
        k2 = (k + 2) % 4
        gwait(k)
        scale(bufs[k], ibufs[k])
        swait(k2)
        if do_gather:
            iwait(c + 2, k2)
            gather(k2)
        for q in range(CHM // 16):
            scolvs[k][pl.ds(q * 16, 16)] = ibufs[k][pl.ds(CHM + q * 16, 16)]
        pltpu.async_copy(bufs[k], ssh.at[scolvs[k]], ssems[k], add=True)
        if do_iload:
            iload(c + 4, k)

    for k in range(4):
        iload(k, k)
    iwait(0, 0)
    gather(0)
    iwait(1, 1)
    gather(1)
    pltpu.async_copy(bufs[2], ssh.at[scolvs[2]], ssems[2], add=True)
    pltpu.async_copy(bufs[3], ssh.at[scolvs[3]], ssems[3], add=True)

    @pl.loop(0, NCHM - 5, step=4)
    def _chunk(c0):
        for q in range(4):
            quarter(c0 + q, q, True, True)

    for c in range(NCHM - 5, NCHM):
        quarter(c, c % 4, c + 2 < NCHM, c + 4 < NCHM)

    swait((NCHM - 2) % 4)
    swait((NCHM - 1) % 4)

    plsc.subcore_barrier()
    nr = RPTM // 128
    remr = RPTM % 128
    for z in range(nr):
        r0 = sid * RPTM + z * 128
        pltpu.async_copy(ssh.at[pl.ds(r0, 128)],
                         out_hbm.at[cid, pl.ds(r0, 128)], isems[0])
    if remr:
        r0 = sid * RPTM + nr * 128
        pltpu.async_copy(ssh.at[pl.ds(r0, remr)],
                         out_hbm.at[cid, pl.ds(r0, remr)], isems[0])
    for z in range(nr):
        r0 = sid * RPTM + z * 128
        pltpu.make_async_copy(ssh.at[pl.ds(r0, 128)],
                              out_hbm.at[cid, pl.ds(r0, 128)], isems[0]).wait()
    if remr:
        r0 = sid * RPTM + nr * 128
        pltpu.make_async_copy(ssh.at[pl.ds(r0, remr)],
                              out_hbm.at[cid, pl.ds(r0, remr)],
                              isems[0]).wait()


def _sc_msg(hp, eib):
    def body(hp_hbm, eib_hbm, out_hbm,
             ib0, ib1, ib2, ib3, sv0, sv1, sv2, sv3, b0, b1, b2, b3, ssh,
             gs0, gs1, gs2, gs3, ss0, ss1, ss2, ss3, is0, is1, is2, is3):
        _sc_msg_body(hp_hbm, eib_hbm, out_hbm,
                     [ib0, ib1, ib2, ib3], [sv0, sv1, sv2, sv3],
                     [b0, b1, b2, b3], ssh,
                     [gs0, gs1, gs2, gs3], [ss0, ss1, ss2, ss3],
                     [is0, is1, is2, is3])

    k = pl.kernel(
        body,
        out_type=jax.ShapeDtypeStruct((NC, NPADM, DD), jnp.float32),
        mesh=_vmesh,
        scratch_types=(
            [pltpu.VMEM((3 * CHM,), jnp.int32) for _ in range(4)]
            + [pltpu.VMEM((CHM,), jnp.int32) for _ in range(4)]
            + [pltpu.VMEM((CHM, DD), jnp.float32) for _ in range(4)]
            + [pltpu.VMEM_SHARED((NPADM, DD), jnp.float32)]
            + [pltpu.SemaphoreType.DMA for _ in range(12)]
        ),
        compiler_params=_sc_params,
    )
    return k(hp, eib)


_RB = 1000
_GRID = NN // _RB


def _tc_first_body(x_ref, w_ref, dgt_ref, hp_ref, dinv_ref):
    deg = 1.0 + dgt_ref[:, 0] + dgt_ref[:, 1]
    dinv = lax.rsqrt(deg)[:, None]
    dinv_ref[...] = dinv
    h = jnp.dot(x_ref[...], w_ref[...], preferred_element_type=jnp.float32)
    hp_ref[...] = h * dinv


def _tc_first(x, W, dgt):
    return pl.pallas_call(
        _tc_first_body,
        grid=(_GRID,),
        in_specs=[
            pl.BlockSpec((_RB, DD), lambda i: (i, 0)),
            pl.BlockSpec((DD, DD), lambda i: (0, 0)),
            pl.BlockSpec((_RB, NC), lambda i: (i, 0)),
        ],
        out_specs=[
            pl.BlockSpec((_RB, DD), lambda i: (i, 0)),
            pl.BlockSpec((_RB, 1), lambda i: (i, 0)),
        ],
        out_shape=[
            jax.ShapeDtypeStruct((NN, DD), jnp.float32),
            jax.ShapeDtypeStruct((NN, 1), jnp.float32),
        ],
    )(x, W, dgt)


def _tc_mid_body(sp_ref, hp_ref, dinv_ref, b_ref, w_ref, out_ref):
    s = sp_ref[0] + sp_ref[1] + hp_ref[...]
    z = jnp.maximum(dinv_ref[...] * s + b_ref[...], 0.0)
    h = jnp.dot(z, w_ref[...], preferred_element_type=jnp.float32)
    out_ref[...] = h * dinv_ref[...]


def _tc_mid(sp, hp, dinv, b, W2):
    return pl.pallas_call(
        _tc_mid_body,
        grid=(_GRID,),
        in_specs=[
            pl.BlockSpec((NC, _RB, DD), lambda i: (0, i, 0)),
            pl.BlockSpec((_RB, DD), lambda i: (i, 0)),
            pl.BlockSpec((_RB, 1), lambda i: (i, 0)),
            pl.BlockSpec((1, DD), lambda i: (0, 0)),
            pl.BlockSpec((DD, DD), lambda i: (0, 0)),
        ],
        out_specs=pl.BlockSpec((_RB, DD), lambda i: (i, 0)),
        out_shape=jax.ShapeDtypeStruct((NN, DD), jnp.float32),
    )(sp, hp, dinv, b, W2)


def _tc_last_body(sp_ref, hp_ref, dinv_ref, b_ref, out_ref):
    s = sp_ref[0] + sp_ref[1] + hp_ref[...]
    out_ref[...] = dinv_ref[...] * s + b_ref[...]


def _tc_last(sp, hp, dinv, b):
    return pl.pallas_call(
        _tc_last_body,
        grid=(_GRID,),
        in_specs=[
            pl.BlockSpec((NC, _RB, DD), lambda i: (0, i, 0)),
            pl.BlockSpec((_RB, DD), lambda i: (i, 0)),
            pl.BlockSpec((_RB, 1), lambda i: (i, 0)),
            pl.BlockSpec((1, DD), lambda i: (0, 0)),
        ],
        out_specs=pl.BlockSpec((_RB, DD), lambda i: (i, 0)),
        out_shape=jax.ShapeDtypeStruct((NN, DD), jnp.float32),
    )(sp, hp, dinv, b)


def kernel(x, edge_index, edge_attr, W1, b1, W2, b2):
    row = edge_index[0]
    col = edge_index[1]
    col3 = col.reshape(NW, NCHUNK, CH)
    ew3 = edge_attr.reshape(NW, NCHUNK, CH)
    ewbits = lax.bitcast_convert_type(edge_attr, jnp.int32)
    eib = jnp.concatenate(
        [row.reshape(NW, NCHM, CHM), col.reshape(NW, NCHM, CHM),
         ewbits.reshape(NW, NCHM, CHM)], axis=2).reshape(-1)
    degp = _sc_deg(col3, ew3)
    dgt = jnp.transpose(degp[:, :NN])
    hp1, dinv = _tc_first(x, W1, dgt)
    s1 = _sc_msg(hp1, eib)
    hp2 = _tc_mid(s1, hp1, dinv, b1.reshape(1, DD), W2)
    s2 = _sc_msg(hp2, eib)
    out = _tc_last(s2, hp2, dinv, b2.reshape(1, DD))
    return out


# speedup vs baseline: 1.0340x; 1.0340x over previous
import dataclasses
import functools

import jax
import jax.numpy as jnp
from jax import lax
from jax.experimental import pallas as pl
from jax.experimental.pallas import tpu as pltpu
from jax.experimental.pallas import tpu_sc as plsc

NN = 10000
EE = 320000
DD = 128
NC = 2
NS = 16
NW = NC * NS
EPW = EE // NW
CH = 80
NCHUNK = EPW // CH
NPAD = 10240
RPT = NPAD // NS
NPADM = 10112
RPTM = NPADM // NS

_vmesh = plsc.VectorSubcoreMesh(core_axis_name="c", subcore_axis_name="s")

_sc_params = pltpu.CompilerParams()
if "needs_layout_passes" in pltpu.CompilerParams.__dataclass_fields__:
    _sc_params = dataclasses.replace(_sc_params, needs_layout_passes=False)


def _splat16(v):
    return jnp.zeros((16,), jnp.int32) + v


def _vcopy_idx(src1d, base, dst1d, n):
    q = 0
    while q + 16 <= n:
        dst1d[pl.ds(q, 16)] = src1d[pl.ds(base + q, 16)]
        q += 16
    if q < n:
        dst1d[pl.ds(n - 16, 16)] = src1d[pl.ds(base + n - 16, 16)]


def _sc_deg_body(col_hbm, ew_hbm, out_hbm, colall, ewall, zv, degsh, sem):
    cid = lax.axis_index("c")
    sid = lax.axis_index("s")
    wid = sid * NC + cid

    @pl.loop(0, RPT, step=16)
    def _zero(i):
        zv[pl.ds(i, 16)] = jnp.zeros((16,), jnp.float32)

    pltpu.sync_copy(zv, degsh.at[pl.ds(sid * RPT, RPT)])
    pltpu.sync_copy(col_hbm.at[wid], colall)
    pltpu.sync_copy(ew_hbm.at[wid], ewall)
    plsc.subcore_barrier()

    @pl.loop(0, NCHUNK // 5)
    def _chunk(g):
        descs = []
        for u in range(5):
            c = g * 5 + u
            descs.append(pltpu.async_copy(
                ewall.at[c], degsh.at[colall.at[c]], sem, add=True))
        for d in descs:
            d.wait()

    plsc.subcore_barrier()
    pltpu.sync_copy(degsh.at[pl.ds(sid * RPT, RPT)],
                    out_hbm.at[cid, pl.ds(sid * RPT, RPT)])


def _sc_deg(col3, ew3):
    k = pl.kernel(
        _sc_deg_body,
        out_type=jax.ShapeDtypeStruct((NC, NPAD), jnp.float32),
        mesh=_vmesh,
        scratch_types=[
            pltpu.VMEM((NCHUNK, CH), jnp.int32),
            pltpu.VMEM((NCHUNK, CH), jnp.float32),
            pltpu.VMEM((RPT,), jnp.float32),
            pltpu.VMEM_SHARED((NPAD,), jnp.float32),
            pltpu.SemaphoreType.DMA,
        ],
        compiler_params=_sc_params,
    )
    return k(col3, ew3)


CHM = 80
NCHM = EPW // CHM


def _sc_msg_body(hp_hbm, row_hbm, col_hbm, ew_hbm, out_hbm,
                 rowvs, colvs, ewvs, scolvs, bufs, ssh, gsems, ssems, isems):
    cid = lax.axis_index("c")
    sid = lax.axis_index("s")
    wid = sid * NC + cid
    base = wid * EPW

    for b in range(4):
        @pl.loop(0, CHM)
        def _zero(r, _b=b):
            for p in range(DD // 16):
                bufs[_b][r, pl.ds(p * 16, 16)] = jnp.zeros((16,), jnp.float32)

        for q in range(CHM // 16):
            scolvs[b][pl.ds(q * 16, 16)] = jnp.zeros((16,), jnp.int32)

    for z in range(RPTM // CHM):
        pltpu.sync_copy(bufs[0], ssh.at[pl.ds(sid * RPTM + z * CHM, CHM)])
    rem = RPTM % CHM
    if rem:
        pltpu.sync_copy(
            bufs[0].at[pl.ds(0, rem)],
            ssh.at[pl.ds(sid * RPTM + (RPTM // CHM) * CHM, rem)])
    plsc.subcore_barrier()

    def iload(c, k):
        sl = pl.ds(base + c * CHM, CHM)
        pltpu.async_copy(row_hbm.at[sl], rowvs[k], isems[k])
        pltpu.async_copy(col_hbm.at[sl], colvs[k], isems[k])
        pltpu.async_copy(ew_hbm.at[sl], ewvs[k], isems[k])

    def iwait(c, k):
        sl = pl.ds(base + c * CHM, CHM)
        pltpu.make_async_copy(row_hbm.at[sl], rowvs[k], isems[k]).wait()
        pltpu.make_async_copy(col_hbm.at[sl], colvs[k], isems[k]).wait()
        pltpu.make_async_copy(ew_hbm.at[sl], ewvs[k], isems[k]).wait()

    def gather(k):
        pltpu.async_copy(hp_hbm.at[rowvs[k]], bufs[k], gsems[k])

    def gwait(k):
        pltpu.make_async_copy(hp_hbm.at[rowvs[k]], bufs[k], gsems[k]).wait()

    def swait(k):
        pltpu.make_async_copy(bufs[k], ssh.at[scolvs[k]], ssems[k]).wait()

    def scale(buf, ewv):
        @pl.loop(0, CHM, step=16)
        def _s(j):
            wv = ewv[pl.ds(j, 16)]
            for u in range(16):
                w = jnp.take(wv, jnp.full((16,), u, jnp.int32))
                for p in range(DD // 16):
                    sl = pl.ds(p * 16, 16)
                    buf[j + u, sl] = buf[j + u, sl] * w

    def quarter(c, k, do_gather, do_iload):
        k2 = (k + 2) % 4
        gwait(k)
        scale(bufs[k], ewvs[k])
        swait(k2)
        if do_gather:
            iwait(c + 2, k2)
            gather(k2)
        _vcopy_idx(colvs[k], 0, scolvs[k], CHM)
        pltpu.async_copy(bufs[k], ssh.at[scolvs[k]], ssems[k], add=True)
        if do_iload:
            iload(c + 4, k)

    for k in range(4):
        iload(k, k)
    iwait(0, 0)
    gather(0)
    iwait(1, 1)
    gather(1)
    pltpu.async_copy(bufs[2], ssh.at[scolvs[2]], ssems[2], add=True)
    pltpu.async_copy(bufs[3], ssh.at[scolvs[3]], ssems[3], add=True)

    @pl.loop(0, NCHM - 5, step=4)
    def _chunk(c0):
        for q in range(4):
            quarter(c0 + q, q, True, True)

    for c in range(NCHM - 5, NCHM):
        quarter(c, c % 4, c + 2 < NCHM, c + 4 < NCHM)

    swait((NCHM - 2) % 4)
    swait((NCHM - 1) % 4)

    plsc.subcore_barrier()
    for z in range(RPTM // 128):
        r0 = sid * RPTM + z * 128
        pltpu.sync_copy(ssh.at[pl.ds(r0, 128)], out_hbm.at[cid, pl.ds(r0, 128)])
    remr = RPTM % 128
    if remr:
        r0 = sid * RPTM + (RPTM // 128) * 128
        pltpu.sync_copy(ssh.at[pl.ds(r0, remr)],
                        out_hbm.at[cid, pl.ds(r0, remr)])


def _sc_msg(hp, row, col, ew):
    def body(hp_hbm, row_hbm, col_hbm, ew_hbm, out_hbm,
             rv0, rv1, rv2, rv3, cv0, cv1, cv2, cv3, ev0, ev1, ev2, ev3,
             sv0, sv1, sv2, sv3, b0, b1, b2, b3, ssh,
             gs0, gs1, gs2, gs3, ss0, ss1, ss2, ss3, is0, is1, is2, is3):
        _sc_msg_body(hp_hbm, row_hbm, col_hbm, ew_hbm, out_hbm,
                     [rv0, rv1, rv2, rv3], [cv0, cv1, cv2, cv3],
                     [ev0, ev1, ev2, ev3], [sv0, sv1, sv2, sv3],
                     [b0, b1, b2, b3], ssh,
                     [gs0, gs1, gs2, gs3], [ss0, ss1, ss2, ss3],
                     [is0, is1, is2, is3])

    k = pl.kernel(
        body,
        out_type=jax.ShapeDtypeStruct((NC, NPADM, DD), jnp.float32),
        mesh=_vmesh,
        scratch_types=(
            [pltpu.VMEM((CHM,), jnp.int32) for _ in range(4)]
            + [pltpu.VMEM((CHM,), jnp.int32) for _ in range(4)]
            + [pltpu.VMEM((CHM,), jnp.float32) for _ in range(4)]
            + [pltpu.VMEM((CHM,), jnp.int32) for _ in range(4)]
            + [pltpu.VMEM((CHM, DD), jnp.float32) for _ in range(4)]
            + [pltpu.VMEM_SHARED((NPADM, DD), jnp.float32)]
            + [pltpu.SemaphoreType.DMA for _ in range(12)]
        ),
        compiler_params=_sc_params,
    )
    return k(hp, row, col, ew)


_RB = 1000
_GRID = NN // _RB


def _tc_first_body(x_ref, w_ref, dgt_ref, hp_ref, dinv_ref):
    deg = 1.0 + dgt_ref[:, 0] + dgt_ref[:, 1]
    dinv = lax.rsqrt(deg)[:, None]
    dinv_ref[...] = dinv
    h = jnp.dot(x_ref[...], w_ref[...], preferred_element_type=jnp.float32)
    hp_ref[...] = h * dinv


def _tc_first(x, W, dgt):
    return pl.pallas_call(
        _tc_first_body,
        grid=(_GRID,),
        in_specs=[
            pl.BlockSpec((_RB, DD), lambda i: (i, 0)),
            pl.BlockSpec((DD, DD), lambda i: (0, 0)),
            pl.BlockSpec((_RB, NC), lambda i: (i, 0)),
        ],
        out_specs=[
            pl.BlockSpec((_RB, DD), lambda i: (i, 0)),
            pl.BlockSpec((_RB, 1), lambda i: (i, 0)),
        ],
        out_shape=[
            jax.ShapeDtypeStruct((NN, DD), jnp.float32),
            jax.ShapeDtypeStruct((NN, 1), jnp.float32),
        ],
    )(x, W, dgt)


def _tc_mid_body(sp_ref, hp_ref, dinv_ref, b_ref, w_ref, out_ref):
    s = sp_ref[0] + sp_ref[1] + hp_ref[...]
    z = jnp.maximum(dinv_ref[...] * s + b_ref[...], 0.0)
    h = jnp.dot(z, w_ref[...], preferred_element_type=jnp.float32)
    out_ref[...] = h * dinv_ref[...]


def _tc_mid(sp, hp, dinv, b, W2):
    return pl.pallas_call(
        _tc_mid_body,
        grid=(_GRID,),
        in_specs=[
            pl.BlockSpec((NC, _RB, DD), lambda i: (0, i, 0)),
            pl.BlockSpec((_RB, DD), lambda i: (i, 0)),
            pl.BlockSpec((_RB, 1), lambda i: (i, 0)),
            pl.BlockSpec((1, DD), lambda i: (0, 0)),
            pl.BlockSpec((DD, DD), lambda i: (0, 0)),
        ],
        out_specs=pl.BlockSpec((_RB, DD), lambda i: (i, 0)),
        out_shape=jax.ShapeDtypeStruct((NN, DD), jnp.float32),
    )(sp, hp, dinv, b, W2)


def _tc_last_body(sp_ref, hp_ref, dinv_ref, b_ref, out_ref):
    s = sp_ref[0] + sp_ref[1] + hp_ref[...]
    out_ref[...] = dinv_ref[...] * s + b_ref[...]


def _tc_last(sp, hp, dinv, b):
    return pl.pallas_call(
        _tc_last_body,
        grid=(_GRID,),
        in_specs=[
            pl.BlockSpec((NC, _RB, DD), lambda i: (0, i, 0)),
            pl.BlockSpec((_RB, DD), lambda i: (i, 0)),
            pl.BlockSpec((_RB, 1), lambda i: (i, 0)),
            pl.BlockSpec((1, DD), lambda i: (0, 0)),
        ],
        out_specs=pl.BlockSpec((_RB, DD), lambda i: (i, 0)),
        out_shape=jax.ShapeDtypeStruct((NN, DD), jnp.float32),
    )(sp, hp, dinv, b)


def kernel(x, edge_index, edge_attr, W1, b1, W2, b2):
    row = edge_index[0]
    col = edge_index[1]
    col3 = col.reshape(NW, NCHUNK, CH)
    ew3 = edge_attr.reshape(NW, NCHUNK, CH)
    degp = _sc_deg(col3, ew3)
    dgt = jnp.transpose(degp[:, :NN])
    hp1, dinv = _tc_first(x, W1, dgt)
    s1 = _sc_msg(hp1, row, col, edge_attr)
    hp2 = _tc_mid(s1, hp1, dinv, b1.reshape(1, DD), W2)
    s2 = _sc_msg(hp2, row, col, edge_attr)
    out = _tc_last(s2, hp2, dinv, b2.reshape(1, DD))
    return out
